# Initial kernel scaffold; baseline (speedup 1.0000x reference)
import jax
import jax.numpy as jnp
from jax.experimental import pallas as pl


def kernel(user_embedding, item_embedding, v_feat, W, b):
    raise NotImplementedError("write your pallas kernel here")


# speedup vs baseline: 1.4312x; 1.4312x over previous
import functools

import jax
import jax.numpy as jnp
from jax.experimental import pallas as pl


def _pick_block(rows_u: int, rows_i: int) -> int:
    for r in (2000, 1000, 800, 500, 200, 100, 40, 8):
        if rows_u % r == 0 and rows_i % r == 0:
            return r
    return 8


def _vbpr_kernel(nu_blocks, user_ref, item_ref, vfeat_ref, w_ref, b_ref, out_ref):
    i = pl.program_id(0)

    @pl.when(i < nu_blocks)
    def _copy_user():
        out_ref[...] = user_ref[...]

    @pl.when(i >= nu_blocks)
    def _item_block():
        visual = jax.lax.dot_general(
            vfeat_ref[...], w_ref[...],
            dimension_numbers=(((1,), (1,)), ((), ())),
            preferred_element_type=jnp.float32,
        ) + b_ref[...]
        out_ref[...] = jnp.concatenate([item_ref[...], visual], axis=-1)


def kernel(user_embedding, item_embedding, v_feat, W, b):
    U, DU = user_embedding.shape
    I, DI = item_embedding.shape
    _, DV = v_feat.shape
    DO = W.shape[0]
    R = _pick_block(U, I)
    nu, ni = U // R, I // R
    b2 = b.reshape(1, DO)

    grid = (nu + ni,)
    out = pl.pallas_call(
        functools.partial(_vbpr_kernel, nu),
        grid=grid,
        in_specs=[
            pl.BlockSpec((R, DU), lambda i: (jnp.minimum(i, nu - 1), 0)),
            pl.BlockSpec((R, DI), lambda i: (jnp.maximum(i - nu, 0), 0)),
            pl.BlockSpec((R, DV), lambda i: (jnp.maximum(i - nu, 0), 0)),
            pl.BlockSpec((DO, DV), lambda i: (0, 0)),
            pl.BlockSpec((1, DO), lambda i: (0, 0)),
        ],
        out_specs=pl.BlockSpec((R, DU), lambda i: (i, 0)),
        out_shape=jax.ShapeDtypeStruct((U + I, DU), user_embedding.dtype),
    )(user_embedding, item_embedding, v_feat, W, b2)
    return out


# speedup vs baseline: 1.6191x; 1.1313x over previous
import functools

import jax
import jax.numpy as jnp
from jax.experimental import pallas as pl


def _pick_block(rows_u: int, rows_i: int) -> int:
    for r in (4000, 2000, 1000, 800, 500, 200, 100, 40, 8):
        if rows_u % r == 0 and rows_i % r == 0:
            return r
    return 8


def _vbpr_kernel(nu_blocks, user_ref, item_ref, vfeat_ref, w_ref, b_ref, out_ref):
    i = pl.program_id(0)

    @pl.when(i < nu_blocks)
    def _copy_user():
        out_ref[...] = user_ref[...]

    @pl.when(i >= nu_blocks)
    def _item_block():
        visual = jax.lax.dot_general(
            vfeat_ref[...], w_ref[...],
            dimension_numbers=(((1,), (1,)), ((), ())),
            preferred_element_type=jnp.float32,
        ) + b_ref[...]
        out_ref[...] = jnp.concatenate([item_ref[...], visual], axis=-1)


def kernel(user_embedding, item_embedding, v_feat, W, b):
    U, DU = user_embedding.shape
    I, DI = item_embedding.shape
    _, DV = v_feat.shape
    DO = W.shape[0]
    R = _pick_block(U, I)
    nu, ni = U // R, I // R
    b2 = b.reshape(1, DO)

    grid = (nu + ni,)
    out = pl.pallas_call(
        functools.partial(_vbpr_kernel, nu),
        grid=grid,
        in_specs=[
            pl.BlockSpec((R, DU), lambda i: (jnp.minimum(i, nu - 1), 0)),
            pl.BlockSpec((R, DI), lambda i: (jnp.maximum(i - nu, 0), 0)),
            pl.BlockSpec((R, DV), lambda i: (jnp.maximum(i - nu, 0), 0)),
            pl.BlockSpec((DO, DV), lambda i: (0, 0)),
            pl.BlockSpec((1, DO), lambda i: (0, 0)),
        ],
        out_specs=pl.BlockSpec((R, DU), lambda i: (i, 0)),
        out_shape=jax.ShapeDtypeStruct((U + I, DU), user_embedding.dtype),
    )(user_embedding, item_embedding, v_feat, W, b2)
    return out


# speedup vs baseline: 1.6316x; 1.0077x over previous
import functools

import jax
import jax.numpy as jnp
from jax.experimental import pallas as pl


def _pick_block(rows_u: int, rows_i: int) -> int:
    for r in (5000, 4000, 2000, 1000, 800, 500, 200, 100, 40, 8):
        if rows_u % r == 0 and rows_i % r == 0:
            return r
    return 8


def _vbpr_kernel(nu_blocks, user_ref, item_ref, vfeat_ref, w_ref, b_ref, out_ref):
    i = pl.program_id(0)

    @pl.when(i < nu_blocks)
    def _copy_user():
        out_ref[...] = user_ref[...]

    @pl.when(i >= nu_blocks)
    def _item_block():
        visual = jax.lax.dot_general(
            vfeat_ref[...], w_ref[...],
            dimension_numbers=(((1,), (1,)), ((), ())),
            preferred_element_type=jnp.float32,
        ) + b_ref[...]
        out_ref[...] = jnp.concatenate([item_ref[...], visual], axis=-1)


def kernel(user_embedding, item_embedding, v_feat, W, b):
    U, DU = user_embedding.shape
    I, DI = item_embedding.shape
    _, DV = v_feat.shape
    DO = W.shape[0]
    R = _pick_block(U, I)
    nu, ni = U // R, I // R
    b2 = b.reshape(1, DO)

    grid = (nu + ni,)
    out = pl.pallas_call(
        functools.partial(_vbpr_kernel, nu),
        grid=grid,
        in_specs=[
            pl.BlockSpec((R, DU), lambda i: (jnp.minimum(i, nu - 1), 0)),
            pl.BlockSpec((R, DI), lambda i: (jnp.maximum(i - nu, 0), 0)),
            pl.BlockSpec((R, DV), lambda i: (jnp.maximum(i - nu, 0), 0)),
            pl.BlockSpec((DO, DV), lambda i: (0, 0)),
            pl.BlockSpec((1, DO), lambda i: (0, 0)),
        ],
        out_specs=pl.BlockSpec((R, DU), lambda i: (i, 0)),
        out_shape=jax.ShapeDtypeStruct((U + I, DU), user_embedding.dtype),
    )(user_embedding, item_embedding, v_feat, W, b2)
    return out
